# Initial kernel scaffold; baseline (speedup 1.0000x reference)
import jax
import jax.numpy as jnp
from jax.experimental import pallas as pl


def kernel(x, edge_index, batch, Wl1, bl1, Wr1, Wl2, bl2, Wr2):
    raise NotImplementedError("write your pallas kernel here")


# speedup vs baseline: 3.4293x; 3.4293x over previous
import jax
import jax.numpy as jnp
from jax import lax
from jax.experimental import pallas as pl
from jax.experimental.pallas import tpu as pltpu
from jax.experimental.pallas import tpu_sc as plsc

N = 10000
E = 320000
D = 128
DH = D // 2
NC = 2
NS = 16
EPT = E // NS
CH = 80
NCH = EPT // CH
NP = 10240
RPT = NP // NS
ZR = 128
CNTW = 16

_f32 = jnp.float32


def _sc_body(xs_hbm, src_hbm, dst_hbm, agg_out, cnt_out,
             src_raw, src_v, dst_v, rows_v, zbuf, zcnt, ebuf, sem,
             agg_sh, cnt_sh):
    c = lax.axis_index("c")
    s = lax.axis_index("s")

    zrow = jnp.zeros((16,), _f32)

    def zb_body(r, carry):
        for j in range(DH // 16):
            zbuf[r, pl.ds(j * 16, 16)] = zrow
        return carry
    lax.fori_loop(0, ZR, zb_body, 0)

    def zc_body(r, carry):
        zcnt[r, :] = zrow
        return carry
    lax.fori_loop(0, RPT, zc_body, 0)

    ehot = jnp.where(lax.iota(jnp.int32, 16) == 0, 1.0, 0.0).astype(_f32)

    def eb_body(r, carry):
        ebuf[r, :] = ehot
        return carry
    lax.fori_loop(0, CH, eb_body, 0)

    rbase = s * RPT
    for j in range(RPT // ZR):
        pltpu.sync_copy(zbuf, agg_sh.at[pl.ds(rbase + j * ZR, ZR)])
    pltpu.sync_copy(zcnt, cnt_sh.at[pl.ds(rbase, RPT)])
    plsc.subcore_barrier()

    ebase = s * EPT
    srcoff = c * N

    def edge_body(i, carry):
        base = pl.multiple_of(ebase + i * CH, 8)
        pltpu.sync_copy(src_hbm.at[pl.ds(base, CH)], src_raw)
        pltpu.sync_copy(dst_hbm.at[pl.ds(base, CH)], dst_v)
        for k in range(CH // 16):
            src_v[pl.ds(k * 16, 16)] = src_raw[pl.ds(k * 16, 16)] + srcoff
        pltpu.async_copy(xs_hbm.at[src_v], rows_v, sem).wait()
        pltpu.sync_copy(rows_v, agg_sh.at[dst_v], add=True)

        @pl.when((i % 2) == c)
        def _():
            pltpu.sync_copy(ebuf, cnt_sh.at[dst_v], add=True)
        return carry
    lax.fori_loop(0, NCH, edge_body, 0)
    plsc.subcore_barrier()

    obase = c * NP + rbase
    pltpu.sync_copy(agg_sh.at[pl.ds(rbase, RPT)], agg_out.at[pl.ds(obase, RPT)])
    pltpu.sync_copy(cnt_sh.at[pl.ds(rbase, RPT)], cnt_out.at[pl.ds(obase, RPT)])


_sc_agg = pl.kernel(
    _sc_body,
    out_type=(
        jax.ShapeDtypeStruct((NC * NP, DH), _f32),
        jax.ShapeDtypeStruct((NC * NP, CNTW), _f32),
    ),
    mesh=plsc.VectorSubcoreMesh(core_axis_name="c", subcore_axis_name="s"),
    compiler_params=pltpu.CompilerParams(use_tc_tiling_on_sc=False),
    scratch_types=[
        pltpu.VMEM((CH,), jnp.int32),
        pltpu.VMEM((CH,), jnp.int32),
        pltpu.VMEM((CH,), jnp.int32),
        pltpu.VMEM((CH, DH), _f32),
        pltpu.VMEM((ZR, DH), _f32),
        pltpu.VMEM((RPT, CNTW), _f32),
        pltpu.VMEM((CH, CNTW), _f32),
        pltpu.SemaphoreType.DMA,
        pltpu.VMEM_SHARED((NP, DH), _f32),
        pltpu.VMEM_SHARED((NP, CNTW), _f32),
    ],
)


def _mean_from_parts(agg_ref, cnt_ref):
    cnt = jnp.sum(cnt_ref[0] + cnt_ref[1], axis=1, keepdims=True)
    inv = 1.0 / jnp.maximum(cnt, 1.0)
    return agg_ref[0] * inv, agg_ref[1] * inv


def _sage_out(m0, m1, xd, wl_ref, bl_ref, wr_ref):
    out = lax.dot_general(m0, wl_ref[:, :DH], (((1,), (1,)), ((), ())),
                          preferred_element_type=_f32)
    out = out + lax.dot_general(m1, wl_ref[:, DH:], (((1,), (1,)), ((), ())),
                                preferred_element_type=_f32)
    out = out + lax.dot_general(xd, wr_ref[...], (((1,), (1,)), ((), ())),
                                preferred_element_type=_f32)
    out = out + bl_ref[...]
    return jnp.maximum(out, 0.0)


def _tc_body1(agg_ref, cnt_ref, x_ref, wl_ref, bl_ref, wr_ref, o_ref):
    m0, m1 = _mean_from_parts(agg_ref, cnt_ref)
    out = _sage_out(m0, m1, x_ref[...], wl_ref, bl_ref, wr_ref)
    o_ref[0] = out[:, :DH]
    o_ref[1] = out[:, DH:]


def _tc_body2(agg_ref, cnt_ref, x_ref, wl_ref, bl_ref, wr_ref, o_ref):
    m0, m1 = _mean_from_parts(agg_ref, cnt_ref)
    xd = jnp.concatenate([x_ref[0], x_ref[1]], axis=1)
    o_ref[...] = _sage_out(m0, m1, xd, wl_ref, bl_ref, wr_ref)


R = 1000


def _tc_layer1(agg, cnt, x, Wl, bl, Wr):
    return pl.pallas_call(
        _tc_body1,
        grid=(N // R,),
        in_specs=[
            pl.BlockSpec((NC, R, DH), lambda i: (0, i, 0)),
            pl.BlockSpec((NC, R, CNTW), lambda i: (0, i, 0)),
            pl.BlockSpec((R, D), lambda i: (i, 0)),
            pl.BlockSpec((D, D), lambda i: (0, 0)),
            pl.BlockSpec((1, D), lambda i: (0, 0)),
            pl.BlockSpec((D, D), lambda i: (0, 0)),
        ],
        out_specs=pl.BlockSpec((NC, R, DH), lambda i: (0, i, 0)),
        out_shape=jax.ShapeDtypeStruct((NC, N, DH), _f32),
    )(agg, cnt, x, Wl, bl, Wr)


def _tc_layer2(agg, cnt, h, Wl, bl, Wr):
    return pl.pallas_call(
        _tc_body2,
        grid=(N // R,),
        in_specs=[
            pl.BlockSpec((NC, R, DH), lambda i: (0, i, 0)),
            pl.BlockSpec((NC, R, CNTW), lambda i: (0, i, 0)),
            pl.BlockSpec((NC, R, DH), lambda i: (0, i, 0)),
            pl.BlockSpec((D, D), lambda i: (0, 0)),
            pl.BlockSpec((1, D), lambda i: (0, 0)),
            pl.BlockSpec((D, D), lambda i: (0, 0)),
        ],
        out_specs=pl.BlockSpec((R, D), lambda i: (i, 0)),
        out_shape=jax.ShapeDtypeStruct((N, D), _f32),
    )(agg, cnt, h, Wl, bl, Wr)


def kernel(x, edge_index, batch, Wl1, bl1, Wr1, Wl2, bl2, Wr2):
    src = edge_index[0]
    dst = edge_index[1]
    xs1 = jnp.moveaxis(x.reshape(N, NC, DH), 1, 0).reshape(NC * N, DH)
    agg1, cnt1 = _sc_agg(xs1, src, dst)
    agg1 = agg1.reshape(NC, NP, DH)
    cnt1 = cnt1.reshape(NC, NP, CNTW)
    h = _tc_layer1(agg1, cnt1, x, Wl1, bl1.reshape(1, D), Wr1)
    agg2, _ = _sc_agg(h.reshape(NC * N, DH), src, dst)
    agg2 = agg2.reshape(NC, NP, DH)
    out = _tc_layer2(agg2, cnt1, h, Wl2, bl2.reshape(1, D), Wr2)
    return out


# speedup vs baseline: 10.5273x; 3.0699x over previous
import jax
import jax.numpy as jnp
from jax import lax
from jax.experimental import pallas as pl
from jax.experimental.pallas import tpu as pltpu
from jax.experimental.pallas import tpu_sc as plsc

N = 10000
E = 320000
D = 128
DH = D // 2
NC = 2
NS = 16
NW = NC * NS
EPT = E // NS
CH = 80
NCH = EPT // CH
CCH = E // (NW * CH)
NBUF = 5
GAHEAD = 2
SLAG = NBUF - GAHEAD
NP = 10240
RPT = NP // NS
ZR = 128
CNTW = 16

_f32 = jnp.float32


def _agg_body(xs_hbm, src_hbm, dst_hbm, agg_out,
              src_all, dst_all, rows0, rows1, rows2, rows3, rows4,
              zbuf, gsem, ssem, psem, agg_sh):
    rows = (rows0, rows1, rows2, rows3, rows4)
    c = lax.axis_index("c")
    s = lax.axis_index("s")

    ibase = s * NCH
    pfs = pltpu.async_copy(src_hbm.at[pl.ds(ibase, NCH)], src_all, psem)
    pfd = pltpu.async_copy(dst_hbm.at[pl.ds(ibase, NCH)], dst_all, psem)

    zrow = jnp.zeros((16,), _f32)

    def zb_body(r, carry):
        for j in range(DH // 16):
            zbuf[r, pl.ds(j * 16, 16)] = zrow
        return carry
    lax.fori_loop(0, ZR, zb_body, 0)

    rbase = s * RPT
    for j in range(RPT // ZR):
        pltpu.sync_copy(zbuf, agg_sh.at[pl.ds(rbase + j * ZR, ZR)])

    pfs.wait()
    pfd.wait()

    srcoff = c * N

    def off_body(r, carry):
        for k in range(CH // 16):
            src_all[r, pl.ds(k * 16, 16)] = (
                src_all[r, pl.ds(k * 16, 16)] + srcoff)
        return carry
    lax.fori_loop(0, NCH, off_body, 0)

    plsc.subcore_barrier()

    def gather_desc(i, b):
        return pltpu.make_async_copy(xs_hbm.at[src_all.at[i]], rows[b], gsem)

    def scatter_desc(i, b):
        return pltpu.make_async_copy(rows[b], agg_sh.at[dst_all.at[i]], ssem)

    for b in range(GAHEAD):
        gather_desc(b, b).start()

    def round_body(j, carry):
        for b in range(NBUF):
            i = j * NBUF + b
            gather_desc(i, b).wait()
            pltpu.async_copy(rows[b], agg_sh.at[dst_all.at[i]], ssem, add=True)

            @pl.when(i >= SLAG)
            def _():
                scatter_desc(i, b).wait()

            @pl.when(i + GAHEAD < NCH)
            def _():
                gather_desc(i + GAHEAD, (b + GAHEAD) % NBUF).start()
        return carry
    lax.fori_loop(0, NCH // NBUF, round_body, 0)

    for _ in range(SLAG):
        scatter_desc(0, 0).wait()

    plsc.subcore_barrier()

    obase = c * NP + rbase
    pltpu.sync_copy(agg_sh.at[pl.ds(rbase, RPT)], agg_out.at[pl.ds(obase, RPT)])


_sc_agg = pl.kernel(
    _agg_body,
    out_type=jax.ShapeDtypeStruct((NC * NP, DH), _f32),
    mesh=plsc.VectorSubcoreMesh(core_axis_name="c", subcore_axis_name="s"),
    compiler_params=pltpu.CompilerParams(use_tc_tiling_on_sc=False),
    scratch_types=(
        [pltpu.VMEM((NCH, CH), jnp.int32),
         pltpu.VMEM((NCH, CH), jnp.int32)]
        + [pltpu.VMEM((CH, DH), _f32) for _ in range(NBUF)]
        + [pltpu.VMEM((ZR, DH), _f32),
           pltpu.SemaphoreType.DMA,
           pltpu.SemaphoreType.DMA,
           pltpu.SemaphoreType.DMA,
           pltpu.VMEM_SHARED((NP, DH), _f32)]
    ),
)


def _cnt_body(dst_hbm, cnt_out, dst_all, zcnt, ebuf, csem, psem, cnt_sh):
    c = lax.axis_index("c")
    s = lax.axis_index("s")
    w = c * NS + s

    pf = pltpu.async_copy(dst_hbm.at[pl.ds(w * CCH, CCH)], dst_all, psem)

    zrow = jnp.zeros((16,), _f32)

    def zc_body(r, carry):
        zcnt[r, :] = zrow
        return carry
    lax.fori_loop(0, RPT, zc_body, 0)

    ehot = jnp.where(lax.iota(jnp.int32, 16) == 0, 1.0, 0.0).astype(_f32)

    def eb_body(r, carry):
        ebuf[r, :] = ehot
        return carry
    lax.fori_loop(0, CH, eb_body, 0)

    rbase = s * RPT
    pltpu.sync_copy(zcnt, cnt_sh.at[pl.ds(rbase, RPT)])
    pf.wait()
    plsc.subcore_barrier()

    def chunk_body(i, carry):
        pltpu.async_copy(ebuf, cnt_sh.at[dst_all.at[i]], csem, add=True)

        @pl.when(i >= 1)
        def _():
            pltpu.make_async_copy(ebuf, cnt_sh.at[dst_all.at[i]], csem).wait()
        return carry
    lax.fori_loop(0, CCH, chunk_body, 0)
    pltpu.make_async_copy(ebuf, cnt_sh.at[dst_all.at[0]], csem).wait()

    plsc.subcore_barrier()
    obase = c * NP + rbase
    pltpu.sync_copy(cnt_sh.at[pl.ds(rbase, RPT)], cnt_out.at[pl.ds(obase, RPT)])


_sc_cnt = pl.kernel(
    _cnt_body,
    out_type=jax.ShapeDtypeStruct((NC * NP, CNTW), _f32),
    mesh=plsc.VectorSubcoreMesh(core_axis_name="c", subcore_axis_name="s"),
    compiler_params=pltpu.CompilerParams(use_tc_tiling_on_sc=False),
    scratch_types=[
        pltpu.VMEM((CCH, CH), jnp.int32),
        pltpu.VMEM((RPT, CNTW), _f32),
        pltpu.VMEM((CH, CNTW), _f32),
        pltpu.SemaphoreType.DMA,
        pltpu.SemaphoreType.DMA,
        pltpu.VMEM_SHARED((NP, CNTW), _f32),
    ],
)


def _mean_from_parts(agg_ref, cnt_ref):
    cnt = jnp.sum(cnt_ref[0] + cnt_ref[1], axis=1, keepdims=True)
    inv = 1.0 / jnp.maximum(cnt, 1.0)
    return agg_ref[0] * inv, agg_ref[1] * inv


def _sage_out(m0, m1, xd, wl_ref, bl_ref, wr_ref):
    out = lax.dot_general(m0, wl_ref[:, :DH], (((1,), (1,)), ((), ())),
                          preferred_element_type=_f32)
    out = out + lax.dot_general(m1, wl_ref[:, DH:], (((1,), (1,)), ((), ())),
                                preferred_element_type=_f32)
    out = out + lax.dot_general(xd, wr_ref[...], (((1,), (1,)), ((), ())),
                                preferred_element_type=_f32)
    out = out + bl_ref[...]
    return jnp.maximum(out, 0.0)


def _tc_body1(agg_ref, cnt_ref, x_ref, wl_ref, bl_ref, wr_ref, o_ref):
    m0, m1 = _mean_from_parts(agg_ref, cnt_ref)
    out = _sage_out(m0, m1, x_ref[...], wl_ref, bl_ref, wr_ref)
    o_ref[0] = out[:, :DH]
    o_ref[1] = out[:, DH:]


def _tc_body2(agg_ref, cnt_ref, x_ref, wl_ref, bl_ref, wr_ref, o_ref):
    m0, m1 = _mean_from_parts(agg_ref, cnt_ref)
    xd = jnp.concatenate([x_ref[0], x_ref[1]], axis=1)
    o_ref[...] = _sage_out(m0, m1, xd, wl_ref, bl_ref, wr_ref)


R = 1000


def _tc_layer1(agg, cnt, x, Wl, bl, Wr):
    return pl.pallas_call(
        _tc_body1,
        grid=(N // R,),
        in_specs=[
            pl.BlockSpec((NC, R, DH), lambda i: (0, i, 0)),
            pl.BlockSpec((NC, R, CNTW), lambda i: (0, i, 0)),
            pl.BlockSpec((R, D), lambda i: (i, 0)),
            pl.BlockSpec((D, D), lambda i: (0, 0)),
            pl.BlockSpec((1, D), lambda i: (0, 0)),
            pl.BlockSpec((D, D), lambda i: (0, 0)),
        ],
        out_specs=pl.BlockSpec((NC, R, DH), lambda i: (0, i, 0)),
        out_shape=jax.ShapeDtypeStruct((NC, N, DH), _f32),
    )(agg, cnt, x, Wl, bl, Wr)


def _tc_layer2(agg, cnt, h, Wl, bl, Wr):
    return pl.pallas_call(
        _tc_body2,
        grid=(N // R,),
        in_specs=[
            pl.BlockSpec((NC, R, DH), lambda i: (0, i, 0)),
            pl.BlockSpec((NC, R, CNTW), lambda i: (0, i, 0)),
            pl.BlockSpec((NC, R, DH), lambda i: (0, i, 0)),
            pl.BlockSpec((D, D), lambda i: (0, 0)),
            pl.BlockSpec((1, D), lambda i: (0, 0)),
            pl.BlockSpec((D, D), lambda i: (0, 0)),
        ],
        out_specs=pl.BlockSpec((R, D), lambda i: (i, 0)),
        out_shape=jax.ShapeDtypeStruct((N, D), _f32),
    )(agg, cnt, h, Wl, bl, Wr)


def kernel(x, edge_index, batch, Wl1, bl1, Wr1, Wl2, bl2, Wr2):
    src = edge_index[0].reshape(E // CH, CH)
    dst = edge_index[1].reshape(E // CH, CH)
    xs1 = jnp.moveaxis(x.reshape(N, NC, DH), 1, 0).reshape(NC * N, DH)
    cnt1 = _sc_cnt(dst).reshape(NC, NP, CNTW)
    agg1 = _sc_agg(xs1, src, dst).reshape(NC, NP, DH)
    h = _tc_layer1(agg1, cnt1, x, Wl1, bl1.reshape(1, D), Wr1)
    agg2 = _sc_agg(h.reshape(NC * N, DH), src, dst).reshape(NC, NP, DH)
    out = _tc_layer2(agg2, cnt1, h, Wl2, bl2.reshape(1, D), Wr2)
    return out


# speedup vs baseline: 11.4604x; 1.0886x over previous
import jax
import jax.numpy as jnp
from jax import lax
from jax.experimental import pallas as pl
from jax.experimental.pallas import tpu as pltpu
from jax.experimental.pallas import tpu_sc as plsc

N = 10000
E = 320000
D = 128
DH = D // 2
NC = 2
NS = 16
NW = NC * NS
EPT = E // NS
CH = 80
NCH = EPT // CH
CCH = E // (NW * CH)
NBUF = 5
GAHEAD = 2
SLAG = NBUF - GAHEAD
NP = 10240
RPT = NP // NS
ZR = 128
CNTW = 16

_f32 = jnp.float32


def _agg_body(xs_hbm, src_hbm, dst_hbm, agg_out,
              src_all, dst_all, rows0, rows1, rows2, rows3, rows4,
              zbuf, gsem, ssem, psem, agg_sh):
    rows = (rows0, rows1, rows2, rows3, rows4)
    c = lax.axis_index("c")
    s = lax.axis_index("s")

    ibase = s * NCH
    pfs = pltpu.async_copy(src_hbm.at[pl.ds(ibase, NCH)], src_all, psem)
    pfd = pltpu.async_copy(dst_hbm.at[pl.ds(ibase, NCH)], dst_all, psem)

    zrow = jnp.zeros((16,), _f32)

    def zb_body(r, carry):
        for j in range(DH // 16):
            zbuf[r, pl.ds(j * 16, 16)] = zrow
        return carry
    lax.fori_loop(0, ZR, zb_body, 0)

    rbase = s * RPT
    for j in range(RPT // ZR):
        pltpu.sync_copy(zbuf, agg_sh.at[pl.ds(rbase + j * ZR, ZR)])

    pfs.wait()
    pfd.wait()

    def off_body(r, carry):
        for k in range(CH // 16):
            src_all[r, pl.ds(k * 16, 16)] = (
                src_all[r, pl.ds(k * 16, 16)] * 2 + c)
        return carry
    lax.fori_loop(0, NCH, off_body, 0)

    plsc.subcore_barrier()

    def gather_desc(i, b):
        return pltpu.make_async_copy(xs_hbm.at[src_all.at[i]], rows[b], gsem)

    def scatter_desc(i, b):
        return pltpu.make_async_copy(rows[b], agg_sh.at[dst_all.at[i]], ssem)

    for b in range(GAHEAD):
        gather_desc(b, b).start()

    def round_body(j, carry):
        for b in range(NBUF):
            i = j * NBUF + b
            gather_desc(i, b).wait()
            pltpu.async_copy(rows[b], agg_sh.at[dst_all.at[i]], ssem, add=True)

            @pl.when(i >= SLAG)
            def _():
                scatter_desc(i, b).wait()

            @pl.when(i + GAHEAD < NCH)
            def _():
                gather_desc(i + GAHEAD, (b + GAHEAD) % NBUF).start()
        return carry
    lax.fori_loop(0, NCH // NBUF, round_body, 0)

    for _ in range(SLAG):
        scatter_desc(0, 0).wait()

    plsc.subcore_barrier()

    obase = c * NP + rbase
    pltpu.sync_copy(agg_sh.at[pl.ds(rbase, RPT)], agg_out.at[pl.ds(obase, RPT)])


_sc_agg = pl.kernel(
    _agg_body,
    out_type=jax.ShapeDtypeStruct((NC * NP, DH), _f32),
    mesh=plsc.VectorSubcoreMesh(core_axis_name="c", subcore_axis_name="s"),
    compiler_params=pltpu.CompilerParams(use_tc_tiling_on_sc=False),
    scratch_types=(
        [pltpu.VMEM((NCH, CH), jnp.int32),
         pltpu.VMEM((NCH, CH), jnp.int32)]
        + [pltpu.VMEM((CH, DH), _f32) for _ in range(NBUF)]
        + [pltpu.VMEM((ZR, DH), _f32),
           pltpu.SemaphoreType.DMA,
           pltpu.SemaphoreType.DMA,
           pltpu.SemaphoreType.DMA,
           pltpu.VMEM_SHARED((NP, DH), _f32)]
    ),
)


def _cnt_body(dst_hbm, cnt_out, dst_all, zcnt, ebuf, csem, psem, cnt_sh):
    c = lax.axis_index("c")
    s = lax.axis_index("s")
    w = c * NS + s

    pf = pltpu.async_copy(dst_hbm.at[pl.ds(w * CCH, CCH)], dst_all, psem)

    zrow = jnp.zeros((16,), _f32)

    def zc_body(r, carry):
        zcnt[r, :] = zrow
        return carry
    lax.fori_loop(0, RPT, zc_body, 0)

    ehot = jnp.where(lax.iota(jnp.int32, 16) == 0, 1.0, 0.0).astype(_f32)

    def eb_body(r, carry):
        ebuf[r, :] = ehot
        return carry
    lax.fori_loop(0, CH, eb_body, 0)

    rbase = s * RPT
    pltpu.sync_copy(zcnt, cnt_sh.at[pl.ds(rbase, RPT)])
    pf.wait()
    plsc.subcore_barrier()

    def chunk_body(i, carry):
        pltpu.async_copy(ebuf, cnt_sh.at[dst_all.at[i]], csem, add=True)

        @pl.when(i >= 1)
        def _():
            pltpu.make_async_copy(ebuf, cnt_sh.at[dst_all.at[i]], csem).wait()
        return carry
    lax.fori_loop(0, CCH, chunk_body, 0)
    pltpu.make_async_copy(ebuf, cnt_sh.at[dst_all.at[0]], csem).wait()

    plsc.subcore_barrier()
    obase = c * NP + rbase
    pltpu.sync_copy(cnt_sh.at[pl.ds(rbase, RPT)], cnt_out.at[pl.ds(obase, RPT)])


_sc_cnt = pl.kernel(
    _cnt_body,
    out_type=jax.ShapeDtypeStruct((NC * NP, CNTW), _f32),
    mesh=plsc.VectorSubcoreMesh(core_axis_name="c", subcore_axis_name="s"),
    compiler_params=pltpu.CompilerParams(use_tc_tiling_on_sc=False),
    scratch_types=[
        pltpu.VMEM((CCH, CH), jnp.int32),
        pltpu.VMEM((RPT, CNTW), _f32),
        pltpu.VMEM((CH, CNTW), _f32),
        pltpu.SemaphoreType.DMA,
        pltpu.SemaphoreType.DMA,
        pltpu.VMEM_SHARED((NP, CNTW), _f32),
    ],
)


def _mean_from_parts(agg_ref, cnt_ref):
    cnt = jnp.sum(cnt_ref[0] + cnt_ref[1], axis=1, keepdims=True)
    inv = 1.0 / jnp.maximum(cnt, 1.0)
    return agg_ref[0] * inv, agg_ref[1] * inv


def _sage_out(m0, m1, xd, wl_ref, bl_ref, wr_ref):
    out = lax.dot_general(m0, wl_ref[:, :DH], (((1,), (1,)), ((), ())),
                          preferred_element_type=_f32)
    out = out + lax.dot_general(m1, wl_ref[:, DH:], (((1,), (1,)), ((), ())),
                                preferred_element_type=_f32)
    out = out + lax.dot_general(xd, wr_ref[...], (((1,), (1,)), ((), ())),
                                preferred_element_type=_f32)
    out = out + bl_ref[...]
    return jnp.maximum(out, 0.0)


def _tc_body(agg_ref, cnt_ref, x_ref, wl_ref, bl_ref, wr_ref, o_ref):
    m0, m1 = _mean_from_parts(agg_ref, cnt_ref)
    o_ref[...] = _sage_out(m0, m1, x_ref[...], wl_ref, bl_ref, wr_ref)


R = 1000


def _tc_layer(agg, cnt, x, Wl, bl, Wr):
    return pl.pallas_call(
        _tc_body,
        grid=(N // R,),
        in_specs=[
            pl.BlockSpec((NC, R, DH), lambda i: (0, i, 0)),
            pl.BlockSpec((NC, R, CNTW), lambda i: (0, i, 0)),
            pl.BlockSpec((R, D), lambda i: (i, 0)),
            pl.BlockSpec((D, D), lambda i: (0, 0)),
            pl.BlockSpec((1, D), lambda i: (0, 0)),
            pl.BlockSpec((D, D), lambda i: (0, 0)),
        ],
        out_specs=pl.BlockSpec((R, D), lambda i: (i, 0)),
        out_shape=jax.ShapeDtypeStruct((N, D), _f32),
    )(agg, cnt, x, Wl, bl, Wr)


def kernel(x, edge_index, batch, Wl1, bl1, Wr1, Wl2, bl2, Wr2):
    src = edge_index[0].reshape(E // CH, CH)
    dst = edge_index[1].reshape(E // CH, CH)
    cnt1 = _sc_cnt(dst).reshape(NC, NP, CNTW)
    agg1 = _sc_agg(x.reshape(NC * N, DH), src, dst).reshape(NC, NP, DH)
    h = _tc_layer(agg1, cnt1, x, Wl1, bl1.reshape(1, D), Wr1)
    agg2 = _sc_agg(h.reshape(NC * N, DH), src, dst).reshape(NC, NP, DH)
    out = _tc_layer(agg2, cnt1, h, Wl2, bl2.reshape(1, D), Wr2)
    return out


# speedup vs baseline: 11.6437x; 1.0160x over previous
import jax
import jax.numpy as jnp
from jax import lax
from jax.experimental import pallas as pl
from jax.experimental.pallas import tpu as pltpu
from jax.experimental.pallas import tpu_sc as plsc

N = 10000
E = 320000
D = 128
DH = D // 2
NC = 2
NS = 16
NW = NC * NS
EPT = E // NS
CH = 80
NCH = EPT // CH
NBUF = 5
GAHEAD = 2
SLAG = NBUF - GAHEAD
NP = 10240
RPT = NP // NS
ZR = 32
ZCR = 160
CNTW = 16

_f32 = jnp.float32


def _make_agg_body(with_cnt):
    def _agg_body(*refs):
        it = iter(refs)
        xs_hbm = next(it); src_hbm = next(it); dst_hbm = next(it)
        agg_out = next(it)
        cnt_out = next(it) if with_cnt else None
        src_all = next(it); dst_all = next(it)
        rows = tuple(next(it) for _ in range(NBUF))
        zbuf = next(it)
        if with_cnt:
            zcnt = next(it); ebuf = next(it)
        gsem = next(it); ssem = next(it)
        csem = next(it) if with_cnt else None
        psem = next(it)
        agg_sh = next(it)
        cnt_sh = next(it) if with_cnt else None
        c = lax.axis_index("c")
        s = lax.axis_index("s")

        ibase = s * NCH
        pfs = pltpu.async_copy(src_hbm.at[pl.ds(ibase, NCH)], src_all, psem)
        pfd = pltpu.async_copy(dst_hbm.at[pl.ds(ibase, NCH)], dst_all, psem)

        zrow = jnp.zeros((16,), _f32)

        def zb_body(r, carry):
            for j in range(DH // 16):
                zbuf[r, pl.ds(j * 16, 16)] = zrow
            return carry
        lax.fori_loop(0, ZR, zb_body, 0)

        rbase = s * RPT
        zd = [pltpu.async_copy(zbuf, agg_sh.at[pl.ds(rbase + j * ZR, ZR)], ssem)
              for j in range(RPT // ZR)]
        if with_cnt:
            def zc_body(r, carry):
                zcnt[r, :] = zrow
                return carry
            lax.fori_loop(0, ZCR, zc_body, 0)

            ehot = jnp.where(lax.iota(jnp.int32, 16) == 0, 1.0, 0.0)

            def eb_body(r, carry):
                ebuf[r, :] = ehot.astype(_f32)
                return carry
            lax.fori_loop(0, CH, eb_body, 0)

            zd += [pltpu.async_copy(
                zcnt, cnt_sh.at[pl.ds(rbase + j * ZCR, ZCR)], ssem)
                for j in range(RPT // ZCR)]
        for d in zd:
            d.wait()

        pfs.wait()
        pfd.wait()

        def off_body(r, carry):
            for k in range(CH // 16):
                src_all[r, pl.ds(k * 16, 16)] = (
                    src_all[r, pl.ds(k * 16, 16)] * 2 + c)
            return carry
        lax.fori_loop(0, NCH, off_body, 0)

        plsc.subcore_barrier()

        def gather_desc(i, b):
            return pltpu.make_async_copy(
                xs_hbm.at[src_all.at[i]], rows[b], gsem)

        def scatter_desc(i, b):
            return pltpu.make_async_copy(
                rows[b], agg_sh.at[dst_all.at[i]], ssem)

        for b in range(GAHEAD):
            gather_desc(b, b).start()

        def round_body(j, carry):
            for b in range(NBUF):
                i = j * NBUF + b
                gather_desc(i, b).wait()
                pltpu.async_copy(rows[b], agg_sh.at[dst_all.at[i]], ssem,
                                 add=True)
                if with_cnt:
                    @pl.when((i % 2) == c)
                    def _():
                        pltpu.async_copy(ebuf, cnt_sh.at[dst_all.at[i]],
                                         csem, add=True)

                        @pl.when(i >= 2)
                        def _():
                            pltpu.make_async_copy(
                                ebuf, cnt_sh.at[dst_all.at[i]], csem).wait()

                @pl.when(i >= SLAG)
                def _():
                    scatter_desc(i, b).wait()

                @pl.when(i + GAHEAD < NCH)
                def _():
                    gather_desc(i + GAHEAD, (b + GAHEAD) % NBUF).start()
            return carry
        lax.fori_loop(0, NCH // NBUF, round_body, 0)

        for _ in range(SLAG):
            scatter_desc(0, 0).wait()
        if with_cnt:
            pltpu.make_async_copy(ebuf, cnt_sh.at[dst_all.at[0]], csem).wait()

        plsc.subcore_barrier()

        obase = c * NP + rbase
        pltpu.sync_copy(agg_sh.at[pl.ds(rbase, RPT)],
                        agg_out.at[pl.ds(obase, RPT)])
        if with_cnt:
            pltpu.sync_copy(cnt_sh.at[pl.ds(rbase, RPT)],
                            cnt_out.at[pl.ds(obase, RPT)])
    return _agg_body


def _build_sc_agg(with_cnt):
    out_type = [jax.ShapeDtypeStruct((NC * NP, DH), _f32)]
    if with_cnt:
        out_type.append(jax.ShapeDtypeStruct((NC * NP, CNTW), _f32))
    scratch = [
        pltpu.VMEM((NCH, CH), jnp.int32),
        pltpu.VMEM((NCH, CH), jnp.int32),
    ]
    scratch += [pltpu.VMEM((CH, DH), _f32) for _ in range(NBUF)]
    scratch += [pltpu.VMEM((ZR, DH), _f32)]
    if with_cnt:
        scratch += [pltpu.VMEM((ZCR, CNTW), _f32),
                    pltpu.VMEM((CH, CNTW), _f32)]
    scratch += [pltpu.SemaphoreType.DMA,
                pltpu.SemaphoreType.DMA]
    if with_cnt:
        scratch += [pltpu.SemaphoreType.DMA]
    scratch += [pltpu.SemaphoreType.DMA,
                pltpu.VMEM_SHARED((NP, DH), _f32)]
    if with_cnt:
        scratch += [pltpu.VMEM_SHARED((NP, CNTW), _f32)]
    return pl.kernel(
        _make_agg_body(with_cnt),
        out_type=tuple(out_type) if with_cnt else out_type[0],
        mesh=plsc.VectorSubcoreMesh(core_axis_name="c", subcore_axis_name="s"),
        compiler_params=pltpu.CompilerParams(use_tc_tiling_on_sc=False),
        scratch_types=scratch,
    )


_sc_agg_cnt = _build_sc_agg(True)
_sc_agg = _build_sc_agg(False)


def _mean_from_parts(agg_ref, cnt_ref):
    cnt = jnp.sum(cnt_ref[0] + cnt_ref[1], axis=1, keepdims=True)
    inv = 1.0 / jnp.maximum(cnt, 1.0)
    return agg_ref[0] * inv, agg_ref[1] * inv


def _sage_out(m0, m1, xd, wl_ref, bl_ref, wr_ref):
    out = lax.dot_general(m0, wl_ref[:, :DH], (((1,), (1,)), ((), ())),
                          preferred_element_type=_f32)
    out = out + lax.dot_general(m1, wl_ref[:, DH:], (((1,), (1,)), ((), ())),
                                preferred_element_type=_f32)
    out = out + lax.dot_general(xd, wr_ref[...], (((1,), (1,)), ((), ())),
                                preferred_element_type=_f32)
    out = out + bl_ref[...]
    return jnp.maximum(out, 0.0)


def _tc_body(agg_ref, cnt_ref, x_ref, wl_ref, bl_ref, wr_ref, o_ref):
    m0, m1 = _mean_from_parts(agg_ref, cnt_ref)
    o_ref[...] = _sage_out(m0, m1, x_ref[...], wl_ref, bl_ref, wr_ref)


R = 1000


def _tc_layer(agg, cnt, x, Wl, bl, Wr):
    return pl.pallas_call(
        _tc_body,
        grid=(N // R,),
        in_specs=[
            pl.BlockSpec((NC, R, DH), lambda i: (0, i, 0)),
            pl.BlockSpec((NC, R, CNTW), lambda i: (0, i, 0)),
            pl.BlockSpec((R, D), lambda i: (i, 0)),
            pl.BlockSpec((D, D), lambda i: (0, 0)),
            pl.BlockSpec((1, D), lambda i: (0, 0)),
            pl.BlockSpec((D, D), lambda i: (0, 0)),
        ],
        out_specs=pl.BlockSpec((R, D), lambda i: (i, 0)),
        out_shape=jax.ShapeDtypeStruct((N, D), _f32),
    )(agg, cnt, x, Wl, bl, Wr)


def kernel(x, edge_index, batch, Wl1, bl1, Wr1, Wl2, bl2, Wr2):
    src = edge_index[0].reshape(E // CH, CH)
    dst = edge_index[1].reshape(E // CH, CH)
    agg1, cnt1 = _sc_agg_cnt(x.reshape(NC * N, DH), src, dst)
    agg1 = agg1.reshape(NC, NP, DH)
    cnt1 = cnt1.reshape(NC, NP, CNTW)
    h = _tc_layer(agg1, cnt1, x, Wl1, bl1.reshape(1, D), Wr1)
    agg2 = _sc_agg(h.reshape(NC * N, DH), src, dst).reshape(NC, NP, DH)
    out = _tc_layer(agg2, cnt1, h, Wl2, bl2.reshape(1, D), Wr2)
    return out


# speedup vs baseline: 11.8731x; 1.0197x over previous
import jax
import jax.numpy as jnp
from jax import lax
from jax.experimental import pallas as pl
from jax.experimental.pallas import tpu as pltpu
from jax.experimental.pallas import tpu_sc as plsc

N = 10000
E = 320000
D = 128
DH = D // 2
NC = 2
NS = 16
NW = NC * NS
EPT = E // NS
CH = 80
NCH = EPT // CH
NBUF = 5
GAHEAD = 2
SLAG = NBUF - GAHEAD
NP = 10240
RPT = NP // NS
ZR = 32
ZCR = 160
CNTW = 16

_f32 = jnp.float32


def _make_agg_body(with_cnt):
    def _agg_body(*refs):
        it = iter(refs)
        xs_hbm = next(it); src_hbm = next(it); dst_hbm = next(it)
        agg_out = next(it)
        cnt_out = next(it) if with_cnt else None
        src_all = next(it); dst_all = next(it)
        rows = tuple(next(it) for _ in range(NBUF))
        zbuf = next(it)
        if with_cnt:
            zcnt = next(it); ebuf = next(it)
        gsem = next(it); ssem = next(it)
        csem = next(it) if with_cnt else None
        psem = next(it)
        agg_sh = next(it)
        cnt_sh = next(it) if with_cnt else None
        c = lax.axis_index("c")
        s = lax.axis_index("s")

        ibase = s * NCH
        pfs = pltpu.async_copy(src_hbm.at[pl.ds(ibase, NCH)], src_all, psem)
        pfd = pltpu.async_copy(dst_hbm.at[pl.ds(ibase, NCH)], dst_all, psem)

        zrow = jnp.zeros((16,), _f32)

        def zb_body(r, carry):
            for j in range(DH // 16):
                zbuf[r, pl.ds(j * 16, 16)] = zrow
            return carry
        lax.fori_loop(0, ZR, zb_body, 0)

        rbase = s * RPT
        zd = [pltpu.async_copy(zbuf, agg_sh.at[pl.ds(rbase + j * ZR, ZR)], ssem)
              for j in range(RPT // ZR)]
        if with_cnt:
            def zc_body(r, carry):
                zcnt[r, :] = zrow
                return carry
            lax.fori_loop(0, ZCR, zc_body, 0)

            ehot = jnp.where(lax.iota(jnp.int32, 16) == 0, 1.0, 0.0)

            def eb_body(r, carry):
                ebuf[r, :] = ehot.astype(_f32)
                return carry
            lax.fori_loop(0, CH, eb_body, 0)

            zd += [pltpu.async_copy(
                zcnt, cnt_sh.at[pl.ds(rbase + j * ZCR, ZCR)], ssem)
                for j in range(RPT // ZCR)]
        for d in zd:
            d.wait()

        pfs.wait()
        pfd.wait()

        def off_body(r, carry):
            for k in range(CH // 16):
                src_all[r, pl.ds(k * 16, 16)] = (
                    src_all[r, pl.ds(k * 16, 16)] * 2 + c)
            return carry
        lax.fori_loop(0, NCH, off_body, 0)

        plsc.subcore_barrier()

        def gather_desc(i, b):
            return pltpu.make_async_copy(
                xs_hbm.at[src_all.at[i]], rows[b], gsem)

        def scatter_desc(i, b):
            return pltpu.make_async_copy(
                rows[b], agg_sh.at[dst_all.at[i]], ssem)

        for b in range(GAHEAD):
            gather_desc(b, b).start()

        def round_body(j, carry):
            for b in range(NBUF):
                i = j * NBUF + b
                gather_desc(i, b).wait()
                pltpu.async_copy(rows[b], agg_sh.at[dst_all.at[i]], ssem,
                                 add=True)
                if with_cnt:
                    @pl.when((i % 2) == c)
                    def _():
                        pltpu.async_copy(ebuf, cnt_sh.at[dst_all.at[i]],
                                         csem, add=True)

                        @pl.when(i >= 2)
                        def _():
                            pltpu.make_async_copy(
                                ebuf, cnt_sh.at[dst_all.at[i]], csem).wait()

                @pl.when(i >= SLAG)
                def _():
                    scatter_desc(i, b).wait()

                @pl.when(i + GAHEAD < NCH)
                def _():
                    gather_desc(i + GAHEAD, (b + GAHEAD) % NBUF).start()
            return carry
        lax.fori_loop(0, NCH // NBUF, round_body, 0)

        for _ in range(SLAG):
            scatter_desc(0, 0).wait()
        if with_cnt:
            pltpu.make_async_copy(ebuf, cnt_sh.at[dst_all.at[0]], csem).wait()

        plsc.subcore_barrier()

        obase = c * NP + rbase
        pltpu.sync_copy(agg_sh.at[pl.ds(rbase, RPT)],
                        agg_out.at[pl.ds(obase, RPT)])
        if with_cnt:
            pltpu.sync_copy(cnt_sh.at[pl.ds(rbase, RPT)],
                            cnt_out.at[pl.ds(obase, RPT)])
    return _agg_body


def _build_sc_agg(with_cnt):
    out_type = [jax.ShapeDtypeStruct((NC * NP, DH), _f32)]
    if with_cnt:
        out_type.append(jax.ShapeDtypeStruct((NC * NP, CNTW), _f32))
    scratch = [
        pltpu.VMEM((NCH, CH), jnp.int32),
        pltpu.VMEM((NCH, CH), jnp.int32),
    ]
    scratch += [pltpu.VMEM((CH, DH), _f32) for _ in range(NBUF)]
    scratch += [pltpu.VMEM((ZR, DH), _f32)]
    if with_cnt:
        scratch += [pltpu.VMEM((ZCR, CNTW), _f32),
                    pltpu.VMEM((CH, CNTW), _f32)]
    scratch += [pltpu.SemaphoreType.DMA,
                pltpu.SemaphoreType.DMA]
    if with_cnt:
        scratch += [pltpu.SemaphoreType.DMA]
    scratch += [pltpu.SemaphoreType.DMA,
                pltpu.VMEM_SHARED((NP, DH), _f32)]
    if with_cnt:
        scratch += [pltpu.VMEM_SHARED((NP, CNTW), _f32)]
    return pl.kernel(
        _make_agg_body(with_cnt),
        out_type=tuple(out_type) if with_cnt else out_type[0],
        mesh=plsc.VectorSubcoreMesh(core_axis_name="c", subcore_axis_name="s"),
        compiler_params=pltpu.CompilerParams(use_tc_tiling_on_sc=False),
        scratch_types=scratch,
    )


_sc_agg_cnt = _build_sc_agg(True)
_sc_agg = _build_sc_agg(False)


def _mean_from_parts(agg_ref, cnt_ref):
    cnt = jnp.sum(cnt_ref[0] + cnt_ref[1], axis=1, keepdims=True)
    inv = 1.0 / jnp.maximum(cnt, 1.0)
    return agg_ref[0] * inv, agg_ref[1] * inv


def _sage_out(m0, m1, xd, wl_ref, bl_ref, wr_ref):
    out = lax.dot_general(m0, wl_ref[:, :DH], (((1,), (1,)), ((), ())),
                          preferred_element_type=_f32)
    out = out + lax.dot_general(m1, wl_ref[:, DH:], (((1,), (1,)), ((), ())),
                                preferred_element_type=_f32)
    out = out + lax.dot_general(xd, wr_ref[...], (((1,), (1,)), ((), ())),
                                preferred_element_type=_f32)
    out = out + bl_ref[...]
    return jnp.maximum(out, 0.0)


def _tc_body(agg_ref, cnt_ref, x_ref, wl_ref, bl_ref, wr_ref, o_ref):
    m0, m1 = _mean_from_parts(agg_ref, cnt_ref)
    o_ref[...] = _sage_out(m0, m1, x_ref[...], wl_ref, bl_ref, wr_ref)


R = 2000


def _tc_layer(agg, cnt, x, Wl, bl, Wr):
    return pl.pallas_call(
        _tc_body,
        grid=(N // R,),
        in_specs=[
            pl.BlockSpec((NC, R, DH), lambda i: (0, i, 0)),
            pl.BlockSpec((NC, R, CNTW), lambda i: (0, i, 0)),
            pl.BlockSpec((R, D), lambda i: (i, 0)),
            pl.BlockSpec((D, D), lambda i: (0, 0)),
            pl.BlockSpec((1, D), lambda i: (0, 0)),
            pl.BlockSpec((D, D), lambda i: (0, 0)),
        ],
        out_specs=pl.BlockSpec((R, D), lambda i: (i, 0)),
        out_shape=jax.ShapeDtypeStruct((N, D), _f32),
    )(agg, cnt, x, Wl, bl, Wr)


def kernel(x, edge_index, batch, Wl1, bl1, Wr1, Wl2, bl2, Wr2):
    src = edge_index[0].reshape(E // CH, CH)
    dst = edge_index[1].reshape(E // CH, CH)
    agg1, cnt1 = _sc_agg_cnt(x.reshape(NC * N, DH), src, dst)
    agg1 = agg1.reshape(NC, NP, DH)
    cnt1 = cnt1.reshape(NC, NP, CNTW)
    h = _tc_layer(agg1, cnt1, x, Wl1, bl1.reshape(1, D), Wr1)
    agg2 = _sc_agg(h.reshape(NC * N, DH), src, dst).reshape(NC, NP, DH)
    out = _tc_layer(agg2, cnt1, h, Wl2, bl2.reshape(1, D), Wr2)
    return out


# speedup vs baseline: 12.5218x; 1.0546x over previous
import jax
import jax.numpy as jnp
from jax import lax
from jax.experimental import pallas as pl
from jax.experimental.pallas import tpu as pltpu
from jax.experimental.pallas import tpu_sc as plsc

N = 10000
E = 320000
D = 128
DH = D // 2
NC = 2
NS = 16
NW = NC * NS
EPT = E // NS
CH = 80
NCH = EPT // CH
NBUF = 5
GAHEAD = 2
SLAG = NBUF - GAHEAD
NP = 10240
RPT = NP // NS
ZR = 32
ZCR = 160
CNTW = 16

_f32 = jnp.float32


def _make_agg_body(with_cnt):
    def _agg_body(*refs):
        it = iter(refs)
        xs_hbm = next(it); src_hbm = next(it); dst_hbm = next(it)
        agg_out = next(it)
        cnt_out = next(it) if with_cnt else None
        src_all = next(it); dst_all = next(it)
        rows = tuple(next(it) for _ in range(NBUF))
        zbuf = next(it)
        if with_cnt:
            zcnt = next(it); ebuf = next(it)
        gsem = next(it); ssem = next(it)
        csem = next(it) if with_cnt else None
        psem = next(it)
        agg_sh = next(it)
        cnt_sh = next(it) if with_cnt else None
        c = lax.axis_index("c")
        s = lax.axis_index("s")

        ibase = s * NCH
        pfs = pltpu.async_copy(src_hbm.at[pl.ds(ibase, NCH)], src_all, psem)
        pfd = pltpu.async_copy(dst_hbm.at[pl.ds(ibase, NCH)], dst_all, psem)

        zrow = jnp.zeros((16,), _f32)

        def zb_body(r, carry):
            for j in range(DH // 16):
                zbuf[r, pl.ds(j * 16, 16)] = zrow
            return carry
        lax.fori_loop(0, ZR, zb_body, 0)

        rbase = s * RPT
        zd = [pltpu.async_copy(zbuf, agg_sh.at[pl.ds(rbase + j * ZR, ZR)], ssem)
              for j in range(RPT // ZR)]
        if with_cnt:
            def zc_body(r, carry):
                zcnt[r, :] = zrow
                return carry
            lax.fori_loop(0, ZCR, zc_body, 0)

            ehot = jnp.where(lax.iota(jnp.int32, 16) == 0, 1.0, 0.0)

            def eb_body(r, carry):
                ebuf[r, :] = ehot.astype(_f32)
                return carry
            lax.fori_loop(0, CH, eb_body, 0)

            zd += [pltpu.async_copy(
                zcnt, cnt_sh.at[pl.ds(rbase + j * ZCR, ZCR)], ssem)
                for j in range(RPT // ZCR)]
        for d in zd:
            d.wait()

        pfs.wait()
        pfd.wait()

        def off_body(r, carry):
            for k in range(CH // 16):
                src_all[r, pl.ds(k * 16, 16)] = (
                    src_all[r, pl.ds(k * 16, 16)] * 2 + c)
            return carry
        lax.fori_loop(0, NCH, off_body, 0)

        plsc.subcore_barrier()

        def gather_desc(i, b):
            return pltpu.make_async_copy(
                xs_hbm.at[src_all.at[i]], rows[b], gsem)

        def scatter_desc(i, b):
            return pltpu.make_async_copy(
                rows[b], agg_sh.at[dst_all.at[i]], ssem)

        for b in range(GAHEAD):
            gather_desc(b, b).start()

        def round_body(j, carry):
            for b in range(NBUF):
                i = j * NBUF + b
                gather_desc(i, b).wait()
                pltpu.async_copy(rows[b], agg_sh.at[dst_all.at[i]], ssem,
                                 add=True)
                if with_cnt:
                    @pl.when((i % 2) == c)
                    def _():
                        pltpu.async_copy(ebuf, cnt_sh.at[dst_all.at[i]],
                                         csem, add=True)

                        @pl.when(i >= 2)
                        def _():
                            pltpu.make_async_copy(
                                ebuf, cnt_sh.at[dst_all.at[i]], csem).wait()

                @pl.when(i >= SLAG)
                def _():
                    scatter_desc(i, b).wait()

                @pl.when(i + GAHEAD < NCH)
                def _():
                    gather_desc(i + GAHEAD, (b + GAHEAD) % NBUF).start()
            return carry
        lax.fori_loop(0, NCH // NBUF, round_body, 0)

        for _ in range(SLAG):
            scatter_desc(0, 0).wait()
        if with_cnt:
            pltpu.make_async_copy(ebuf, cnt_sh.at[dst_all.at[0]], csem).wait()

        plsc.subcore_barrier()

        obase = c * NP + rbase
        pltpu.sync_copy(agg_sh.at[pl.ds(rbase, RPT)],
                        agg_out.at[pl.ds(obase, RPT), pl.ds(0, DH)])
        if with_cnt:
            pltpu.sync_copy(cnt_sh.at[pl.ds(rbase, RPT)],
                            cnt_out.at[pl.ds(obase, RPT)])
    return _agg_body


def _build_sc_agg(with_cnt):
    out_type = [jax.ShapeDtypeStruct((NC * NP, D), _f32)]
    if with_cnt:
        out_type.append(jax.ShapeDtypeStruct((NC * NP, CNTW), _f32))
    scratch = [
        pltpu.VMEM((NCH, CH), jnp.int32),
        pltpu.VMEM((NCH, CH), jnp.int32),
    ]
    scratch += [pltpu.VMEM((CH, DH), _f32) for _ in range(NBUF)]
    scratch += [pltpu.VMEM((ZR, DH), _f32)]
    if with_cnt:
        scratch += [pltpu.VMEM((ZCR, CNTW), _f32),
                    pltpu.VMEM((CH, CNTW), _f32)]
    scratch += [pltpu.SemaphoreType.DMA,
                pltpu.SemaphoreType.DMA]
    if with_cnt:
        scratch += [pltpu.SemaphoreType.DMA]
    scratch += [pltpu.SemaphoreType.DMA,
                pltpu.VMEM_SHARED((NP, DH), _f32)]
    if with_cnt:
        scratch += [pltpu.VMEM_SHARED((NP, CNTW), _f32)]
    return pl.kernel(
        _make_agg_body(with_cnt),
        out_type=tuple(out_type) if with_cnt else out_type[0],
        mesh=plsc.VectorSubcoreMesh(core_axis_name="c", subcore_axis_name="s"),
        compiler_params=pltpu.CompilerParams(use_tc_tiling_on_sc=False),
        scratch_types=scratch,
    )


_sc_agg_cnt = _build_sc_agg(True)
_sc_agg = _build_sc_agg(False)


def _mean_from_parts(agg_ref, cnt_ref):
    cnt = jnp.sum(cnt_ref[0] + cnt_ref[1], axis=1, keepdims=True)
    inv = 1.0 / jnp.maximum(cnt, 1.0)
    return agg_ref[0, :, :DH] * inv, agg_ref[1, :, :DH] * inv


def _sage_out(m0, m1, xd, wl_ref, bl_ref, wr_ref):
    out = lax.dot_general(m0, wl_ref[:, :DH], (((1,), (1,)), ((), ())),
                          preferred_element_type=_f32)
    out = out + lax.dot_general(m1, wl_ref[:, DH:], (((1,), (1,)), ((), ())),
                                preferred_element_type=_f32)
    out = out + lax.dot_general(xd, wr_ref[...], (((1,), (1,)), ((), ())),
                                preferred_element_type=_f32)
    out = out + bl_ref[...]
    return jnp.maximum(out, 0.0)


def _tc_body(agg_ref, cnt_ref, x_ref, wl_ref, bl_ref, wr_ref, o_ref):
    m0, m1 = _mean_from_parts(agg_ref, cnt_ref)
    o_ref[...] = _sage_out(m0, m1, x_ref[...], wl_ref, bl_ref, wr_ref)


R = 2000


def _tc_layer(agg, cnt, x, Wl, bl, Wr):
    return pl.pallas_call(
        _tc_body,
        grid=(N // R,),
        in_specs=[
            pl.BlockSpec((NC, R, D), lambda i: (0, i, 0)),
            pl.BlockSpec((NC, R, CNTW), lambda i: (0, i, 0)),
            pl.BlockSpec((R, D), lambda i: (i, 0)),
            pl.BlockSpec((D, D), lambda i: (0, 0)),
            pl.BlockSpec((1, D), lambda i: (0, 0)),
            pl.BlockSpec((D, D), lambda i: (0, 0)),
        ],
        out_specs=pl.BlockSpec((R, D), lambda i: (i, 0)),
        out_shape=jax.ShapeDtypeStruct((N, D), _f32),
    )(agg, cnt, x, Wl, bl, Wr)


def kernel(x, edge_index, batch, Wl1, bl1, Wr1, Wl2, bl2, Wr2):
    src = edge_index[0].reshape(E // CH, CH)
    dst = edge_index[1].reshape(E // CH, CH)
    agg1, cnt1 = _sc_agg_cnt(x.reshape(NC * N, DH), src, dst)
    agg1 = agg1.reshape(NC, NP, D)
    cnt1 = cnt1.reshape(NC, NP, CNTW)
    h = _tc_layer(agg1, cnt1, x, Wl1, bl1.reshape(1, D), Wr1)
    agg2 = _sc_agg(h.reshape(NC * N, DH), src, dst).reshape(NC, NP, D)
    out = _tc_layer(agg2, cnt1, h, Wl2, bl2.reshape(1, D), Wr2)
    return out


# speedup vs baseline: 15.1662x; 1.2112x over previous
import jax
import jax.numpy as jnp
from jax import lax
from jax.experimental import pallas as pl
from jax.experimental.pallas import tpu as pltpu
from jax.experimental.pallas import tpu_sc as plsc

N = 10000
E = 320000
D = 128
DH = D // 2
NC = 2
NS = 16
NW = NC * NS
EPT = E // NS
CH = 80
NCH = EPT // CH
NBUF = 5
GAHEAD = 3
SLAG = NBUF - GAHEAD
NP = 10240
RPT = NP // NS
ZR = 32
ZCR = 160
CNTW = 16

_f32 = jnp.float32


def _make_agg_body(with_cnt):
    def _agg_body(*refs):
        it = iter(refs)
        xs_hbm = next(it); src_hbm = next(it); dst_hbm = next(it)
        agg_out = next(it)
        cnt_out = next(it) if with_cnt else None
        src_all = next(it); dst_all = next(it)
        rows = tuple(next(it) for _ in range(NBUF))
        zbuf = next(it)
        if with_cnt:
            zcnt = next(it); ebuf = next(it)
        gsem = next(it); ssem = next(it)
        csem = next(it) if with_cnt else None
        psem = next(it)
        agg_sh = next(it)
        cnt_sh = next(it) if with_cnt else None
        c = lax.axis_index("c")
        s = lax.axis_index("s")

        ibase = s * NCH
        pfs = pltpu.async_copy(src_hbm.at[pl.ds(ibase, NCH)], src_all, psem)
        pfd = pltpu.async_copy(dst_hbm.at[pl.ds(ibase, NCH)], dst_all, psem)

        zrow = jnp.zeros((16,), _f32)

        def zb_body(r, carry):
            for j in range(DH // 16):
                zbuf[r, pl.ds(j * 16, 16)] = zrow
            return carry
        lax.fori_loop(0, ZR, zb_body, 0)

        rbase = s * RPT
        zd = [pltpu.async_copy(zbuf, agg_sh.at[pl.ds(rbase + j * ZR, ZR)], ssem)
              for j in range(RPT // ZR)]
        if with_cnt:
            def zc_body(r, carry):
                zcnt[r, :] = zrow
                return carry
            lax.fori_loop(0, ZCR, zc_body, 0)

            ehot = jnp.where(lax.iota(jnp.int32, 16) == 0, 1.0, 0.0)

            def eb_body(r, carry):
                ebuf[r, :] = ehot.astype(_f32)
                return carry
            lax.fori_loop(0, CH, eb_body, 0)

            zd += [pltpu.async_copy(
                zcnt, cnt_sh.at[pl.ds(rbase + j * ZCR, ZCR)], ssem)
                for j in range(RPT // ZCR)]
        for d in zd:
            d.wait()

        pfs.wait()
        pfd.wait()

        def off_body(r, carry):
            for k in range(CH // 16):
                src_all[r, pl.ds(k * 16, 16)] = (
                    src_all[r, pl.ds(k * 16, 16)] * 2 + c)
            return carry
        lax.fori_loop(0, NCH, off_body, 0)

        plsc.subcore_barrier()

        def gather_desc(i, b):
            return pltpu.make_async_copy(
                xs_hbm.at[src_all.at[i]], rows[b], gsem)

        def scatter_desc(i, b):
            return pltpu.make_async_copy(
                rows[b], agg_sh.at[dst_all.at[i]], ssem)

        for b in range(GAHEAD):
            gather_desc(b, b).start()

        def round_body(j, carry):
            for b in range(NBUF):
                i = j * NBUF + b
                gather_desc(i, b).wait()
                pltpu.async_copy(rows[b], agg_sh.at[dst_all.at[i]], ssem,
                                 add=True)
                if with_cnt:
                    @pl.when((i % 2) == c)
                    def _():
                        pltpu.async_copy(ebuf, cnt_sh.at[dst_all.at[i]],
                                         csem, add=True)

                        @pl.when(i >= 2)
                        def _():
                            pltpu.make_async_copy(
                                ebuf, cnt_sh.at[dst_all.at[i]], csem).wait()

                @pl.when(i >= SLAG)
                def _():
                    scatter_desc(i, b).wait()

                @pl.when(i + GAHEAD < NCH)
                def _():
                    gather_desc(i + GAHEAD, (b + GAHEAD) % NBUF).start()
            return carry
        lax.fori_loop(0, NCH // NBUF, round_body, 0)

        for _ in range(SLAG):
            scatter_desc(0, 0).wait()
        if with_cnt:
            pltpu.make_async_copy(ebuf, cnt_sh.at[dst_all.at[0]], csem).wait()

        plsc.subcore_barrier()

        obase = c * NP + rbase
        pltpu.sync_copy(agg_sh.at[pl.ds(rbase, RPT)],
                        agg_out.at[pl.ds(obase, RPT), pl.ds(0, DH)])
        if with_cnt:
            pltpu.sync_copy(cnt_sh.at[pl.ds(rbase, RPT)],
                            cnt_out.at[pl.ds(obase, RPT)])
    return _agg_body


def _build_sc_agg(with_cnt):
    out_type = [jax.ShapeDtypeStruct((NC * NP, D), _f32)]
    if with_cnt:
        out_type.append(jax.ShapeDtypeStruct((NC * NP, CNTW), _f32))
    scratch = [
        pltpu.VMEM((NCH, CH), jnp.int32),
        pltpu.VMEM((NCH, CH), jnp.int32),
    ]
    scratch += [pltpu.VMEM((CH, DH), _f32) for _ in range(NBUF)]
    scratch += [pltpu.VMEM((ZR, DH), _f32)]
    if with_cnt:
        scratch += [pltpu.VMEM((ZCR, CNTW), _f32),
                    pltpu.VMEM((CH, CNTW), _f32)]
    scratch += [pltpu.SemaphoreType.DMA,
                pltpu.SemaphoreType.DMA]
    if with_cnt:
        scratch += [pltpu.SemaphoreType.DMA]
    scratch += [pltpu.SemaphoreType.DMA,
                pltpu.VMEM_SHARED((NP, DH), _f32)]
    if with_cnt:
        scratch += [pltpu.VMEM_SHARED((NP, CNTW), _f32)]
    return pl.kernel(
        _make_agg_body(with_cnt),
        out_type=tuple(out_type) if with_cnt else out_type[0],
        mesh=plsc.VectorSubcoreMesh(core_axis_name="c", subcore_axis_name="s"),
        compiler_params=pltpu.CompilerParams(use_tc_tiling_on_sc=False),
        scratch_types=scratch,
    )


_sc_agg_cnt = _build_sc_agg(True)
_sc_agg = _build_sc_agg(False)


def _mean_from_parts(agg_ref, cnt_ref):
    cnt = jnp.sum(cnt_ref[0] + cnt_ref[1], axis=1, keepdims=True)
    inv = 1.0 / jnp.maximum(cnt, 1.0)
    return agg_ref[0, :, :DH] * inv, agg_ref[1, :, :DH] * inv


def _sage_out(m0, m1, xd, wl_ref, bl_ref, wr_ref):
    out = lax.dot_general(m0, wl_ref[:, :DH], (((1,), (1,)), ((), ())),
                          preferred_element_type=_f32)
    out = out + lax.dot_general(m1, wl_ref[:, DH:], (((1,), (1,)), ((), ())),
                                preferred_element_type=_f32)
    out = out + lax.dot_general(xd, wr_ref[...], (((1,), (1,)), ((), ())),
                                preferred_element_type=_f32)
    out = out + bl_ref[...]
    return jnp.maximum(out, 0.0)


def _tc_body(agg_ref, cnt_ref, x_ref, wl_ref, bl_ref, wr_ref, o_ref):
    m0, m1 = _mean_from_parts(agg_ref, cnt_ref)
    o_ref[...] = _sage_out(m0, m1, x_ref[...], wl_ref, bl_ref, wr_ref)


R = 2000


def _tc_layer(agg, cnt, x, Wl, bl, Wr):
    return pl.pallas_call(
        _tc_body,
        grid=(N // R,),
        in_specs=[
            pl.BlockSpec((NC, R, D), lambda i: (0, i, 0)),
            pl.BlockSpec((NC, R, CNTW), lambda i: (0, i, 0)),
            pl.BlockSpec((R, D), lambda i: (i, 0)),
            pl.BlockSpec((D, D), lambda i: (0, 0)),
            pl.BlockSpec((1, D), lambda i: (0, 0)),
            pl.BlockSpec((D, D), lambda i: (0, 0)),
        ],
        out_specs=pl.BlockSpec((R, D), lambda i: (i, 0)),
        out_shape=jax.ShapeDtypeStruct((N, D), _f32),
    )(agg, cnt, x, Wl, bl, Wr)


def kernel(x, edge_index, batch, Wl1, bl1, Wr1, Wl2, bl2, Wr2):
    src = edge_index[0].reshape(E // CH, CH)
    dst = edge_index[1].reshape(E // CH, CH)
    agg1, cnt1 = _sc_agg_cnt(x.reshape(NC * N, DH), src, dst)
    agg1 = agg1.reshape(NC, NP, D)
    cnt1 = cnt1.reshape(NC, NP, CNTW)
    h = _tc_layer(agg1, cnt1, x, Wl1, bl1.reshape(1, D), Wr1)
    agg2 = _sc_agg(h.reshape(NC * N, DH), src, dst).reshape(NC, NP, D)
    out = _tc_layer(agg2, cnt1, h, Wl2, bl2.reshape(1, D), Wr2)
    return out


# speedup vs baseline: 15.9609x; 1.0524x over previous
import jax
import jax.numpy as jnp
from jax import lax
from jax.experimental import pallas as pl
from jax.experimental.pallas import tpu as pltpu
from jax.experimental.pallas import tpu_sc as plsc

N = 10000
E = 320000
D = 128
DH = D // 2
NC = 2
NS = 16
NW = NC * NS
EPT = E // NS
CH = 80
NCH = EPT // CH
NBUF = 5
GAHEAD = 4
SLAG = NBUF - GAHEAD
NP = 10240
RPT = NP // NS
ZR = 32
ZCR = 160
CNTW = 16

_f32 = jnp.float32


def _make_agg_body(with_cnt):
    def _agg_body(*refs):
        it = iter(refs)
        xs_hbm = next(it); src_hbm = next(it); dst_hbm = next(it)
        agg_out = next(it)
        cnt_out = next(it) if with_cnt else None
        src_all = next(it); dst_all = next(it)
        rows = tuple(next(it) for _ in range(NBUF))
        zbuf = next(it)
        if with_cnt:
            zcnt = next(it); ebuf = next(it)
        gsem = next(it); ssem = next(it)
        csem = next(it) if with_cnt else None
        psem = next(it)
        agg_sh = next(it)
        cnt_sh = next(it) if with_cnt else None
        c = lax.axis_index("c")
        s = lax.axis_index("s")

        ibase = s * NCH
        pfs = pltpu.async_copy(src_hbm.at[pl.ds(ibase, NCH)], src_all, psem)
        pfd = pltpu.async_copy(dst_hbm.at[pl.ds(ibase, NCH)], dst_all, psem)

        zrow = jnp.zeros((16,), _f32)

        def zb_body(r, carry):
            for j in range(DH // 16):
                zbuf[r, pl.ds(j * 16, 16)] = zrow
            return carry
        lax.fori_loop(0, ZR, zb_body, 0)

        rbase = s * RPT
        zd = [pltpu.async_copy(zbuf, agg_sh.at[pl.ds(rbase + j * ZR, ZR)], ssem)
              for j in range(RPT // ZR)]
        if with_cnt:
            def zc_body(r, carry):
                zcnt[r, :] = zrow
                return carry
            lax.fori_loop(0, ZCR, zc_body, 0)

            ehot = jnp.where(lax.iota(jnp.int32, 16) == 0, 1.0, 0.0)

            def eb_body(r, carry):
                ebuf[r, :] = ehot.astype(_f32)
                return carry
            lax.fori_loop(0, CH, eb_body, 0)

            zd += [pltpu.async_copy(
                zcnt, cnt_sh.at[pl.ds(rbase + j * ZCR, ZCR)], ssem)
                for j in range(RPT // ZCR)]
        for d in zd:
            d.wait()

        pfs.wait()
        pfd.wait()

        def off_body(r, carry):
            for k in range(CH // 16):
                src_all[r, pl.ds(k * 16, 16)] = (
                    src_all[r, pl.ds(k * 16, 16)] * 2 + c)
            return carry
        lax.fori_loop(0, NCH, off_body, 0)

        plsc.subcore_barrier()

        def gather_desc(i, b):
            return pltpu.make_async_copy(
                xs_hbm.at[src_all.at[i]], rows[b], gsem)

        def scatter_desc(i, b):
            return pltpu.make_async_copy(
                rows[b], agg_sh.at[dst_all.at[i]], ssem)

        for b in range(GAHEAD):
            gather_desc(b, b).start()

        def round_body(j, carry):
            for b in range(NBUF):
                i = j * NBUF + b
                gather_desc(i, b).wait()
                pltpu.async_copy(rows[b], agg_sh.at[dst_all.at[i]], ssem,
                                 add=True)
                if with_cnt:
                    @pl.when((i % 2) == c)
                    def _():
                        pltpu.async_copy(ebuf, cnt_sh.at[dst_all.at[i]],
                                         csem, add=True)

                        @pl.when(i >= 2)
                        def _():
                            pltpu.make_async_copy(
                                ebuf, cnt_sh.at[dst_all.at[i]], csem).wait()

                @pl.when(i >= SLAG)
                def _():
                    scatter_desc(i, b).wait()

                @pl.when(i + GAHEAD < NCH)
                def _():
                    gather_desc(i + GAHEAD, (b + GAHEAD) % NBUF).start()
            return carry
        lax.fori_loop(0, NCH // NBUF, round_body, 0)

        for _ in range(SLAG):
            scatter_desc(0, 0).wait()
        if with_cnt:
            pltpu.make_async_copy(ebuf, cnt_sh.at[dst_all.at[0]], csem).wait()

        plsc.subcore_barrier()

        obase = c * NP + rbase
        pltpu.sync_copy(agg_sh.at[pl.ds(rbase, RPT)],
                        agg_out.at[pl.ds(obase, RPT), pl.ds(0, DH)])
        if with_cnt:
            pltpu.sync_copy(cnt_sh.at[pl.ds(rbase, RPT)],
                            cnt_out.at[pl.ds(obase, RPT)])
    return _agg_body


def _build_sc_agg(with_cnt):
    out_type = [jax.ShapeDtypeStruct((NC * NP, D), _f32)]
    if with_cnt:
        out_type.append(jax.ShapeDtypeStruct((NC * NP, CNTW), _f32))
    scratch = [
        pltpu.VMEM((NCH, CH), jnp.int32),
        pltpu.VMEM((NCH, CH), jnp.int32),
    ]
    scratch += [pltpu.VMEM((CH, DH), _f32) for _ in range(NBUF)]
    scratch += [pltpu.VMEM((ZR, DH), _f32)]
    if with_cnt:
        scratch += [pltpu.VMEM((ZCR, CNTW), _f32),
                    pltpu.VMEM((CH, CNTW), _f32)]
    scratch += [pltpu.SemaphoreType.DMA,
                pltpu.SemaphoreType.DMA]
    if with_cnt:
        scratch += [pltpu.SemaphoreType.DMA]
    scratch += [pltpu.SemaphoreType.DMA,
                pltpu.VMEM_SHARED((NP, DH), _f32)]
    if with_cnt:
        scratch += [pltpu.VMEM_SHARED((NP, CNTW), _f32)]
    return pl.kernel(
        _make_agg_body(with_cnt),
        out_type=tuple(out_type) if with_cnt else out_type[0],
        mesh=plsc.VectorSubcoreMesh(core_axis_name="c", subcore_axis_name="s"),
        compiler_params=pltpu.CompilerParams(use_tc_tiling_on_sc=False),
        scratch_types=scratch,
    )


_sc_agg_cnt = _build_sc_agg(True)
_sc_agg = _build_sc_agg(False)


def _mean_from_parts(agg_ref, cnt_ref):
    cnt = jnp.sum(cnt_ref[0] + cnt_ref[1], axis=1, keepdims=True)
    inv = 1.0 / jnp.maximum(cnt, 1.0)
    return agg_ref[0, :, :DH] * inv, agg_ref[1, :, :DH] * inv


def _sage_out(m0, m1, xd, wl_ref, bl_ref, wr_ref):
    out = lax.dot_general(m0, wl_ref[:, :DH], (((1,), (1,)), ((), ())),
                          preferred_element_type=_f32)
    out = out + lax.dot_general(m1, wl_ref[:, DH:], (((1,), (1,)), ((), ())),
                                preferred_element_type=_f32)
    out = out + lax.dot_general(xd, wr_ref[...], (((1,), (1,)), ((), ())),
                                preferred_element_type=_f32)
    out = out + bl_ref[...]
    return jnp.maximum(out, 0.0)


def _tc_body(agg_ref, cnt_ref, x_ref, wl_ref, bl_ref, wr_ref, o_ref):
    m0, m1 = _mean_from_parts(agg_ref, cnt_ref)
    o_ref[...] = _sage_out(m0, m1, x_ref[...], wl_ref, bl_ref, wr_ref)


R = 2000


def _tc_layer(agg, cnt, x, Wl, bl, Wr):
    return pl.pallas_call(
        _tc_body,
        grid=(N // R,),
        in_specs=[
            pl.BlockSpec((NC, R, D), lambda i: (0, i, 0)),
            pl.BlockSpec((NC, R, CNTW), lambda i: (0, i, 0)),
            pl.BlockSpec((R, D), lambda i: (i, 0)),
            pl.BlockSpec((D, D), lambda i: (0, 0)),
            pl.BlockSpec((1, D), lambda i: (0, 0)),
            pl.BlockSpec((D, D), lambda i: (0, 0)),
        ],
        out_specs=pl.BlockSpec((R, D), lambda i: (i, 0)),
        out_shape=jax.ShapeDtypeStruct((N, D), _f32),
    )(agg, cnt, x, Wl, bl, Wr)


def kernel(x, edge_index, batch, Wl1, bl1, Wr1, Wl2, bl2, Wr2):
    src = edge_index[0].reshape(E // CH, CH)
    dst = edge_index[1].reshape(E // CH, CH)
    agg1, cnt1 = _sc_agg_cnt(x.reshape(NC * N, DH), src, dst)
    agg1 = agg1.reshape(NC, NP, D)
    cnt1 = cnt1.reshape(NC, NP, CNTW)
    h = _tc_layer(agg1, cnt1, x, Wl1, bl1.reshape(1, D), Wr1)
    agg2 = _sc_agg(h.reshape(NC * N, DH), src, dst).reshape(NC, NP, D)
    out = _tc_layer(agg2, cnt1, h, Wl2, bl2.reshape(1, D), Wr2)
    return out


# speedup vs baseline: 16.2471x; 1.0179x over previous
import jax
import jax.numpy as jnp
from jax import lax
from jax.experimental import pallas as pl
from jax.experimental.pallas import tpu as pltpu
from jax.experimental.pallas import tpu_sc as plsc

N = 10000
E = 320000
D = 128
DH = D // 2
NC = 2
NS = 16
NW = NC * NS
EPT = E // NS
CH = 80
NCH = EPT // CH
NBUF = 5
GAHEAD = 4
SLAG = NBUF - GAHEAD
NP = 10240
RPT = NP // NS
ZR = 32
ZCR = 160
CNTW = 16

_f32 = jnp.float32


def _make_agg_body(with_cnt):
    def _agg_body(*refs):
        it = iter(refs)
        xs_hbm = next(it); src_hbm = next(it); dst_hbm = next(it)
        agg_out = next(it)
        cnt_out = next(it) if with_cnt else None
        src_all = next(it); dst_all = next(it)
        rows = tuple(next(it) for _ in range(NBUF))
        zbuf = next(it)
        if with_cnt:
            zcnt = next(it); ebuf = next(it)
        gsem = next(it); ssem = next(it)
        csem = next(it) if with_cnt else None
        psem = next(it)
        agg_sh = next(it)
        cnt_sh = next(it) if with_cnt else None
        c = lax.axis_index("c")
        s = lax.axis_index("s")

        ibase = s * NCH
        pfs = pltpu.async_copy(src_hbm.at[pl.ds(ibase, NCH)], src_all, psem)
        pfd = pltpu.async_copy(dst_hbm.at[pl.ds(ibase, NCH)], dst_all, psem)

        zrow = jnp.zeros((16,), _f32)

        def zb_body(r, carry):
            for j in range(DH // 16):
                zbuf[r, pl.ds(j * 16, 16)] = zrow
            return carry
        lax.fori_loop(0, ZR, zb_body, 0)

        rbase = s * RPT
        zd = [pltpu.async_copy(zbuf, agg_sh.at[pl.ds(rbase + j * ZR, ZR)], ssem)
              for j in range(RPT // ZR)]
        if with_cnt:
            def zc_body(r, carry):
                zcnt[r, :] = zrow
                return carry
            lax.fori_loop(0, ZCR, zc_body, 0)

            ehot = jnp.where(lax.iota(jnp.int32, 16) == 0, 1.0, 0.0)

            def eb_body(r, carry):
                ebuf[r, :] = ehot.astype(_f32)
                return carry
            lax.fori_loop(0, CH, eb_body, 0)

            zd += [pltpu.async_copy(
                zcnt, cnt_sh.at[pl.ds(rbase + j * ZCR, ZCR)], ssem)
                for j in range(RPT // ZCR)]
        for d in zd:
            d.wait()

        pfs.wait()
        pfd.wait()

        def off_body(r, carry):
            for k in range(CH // 16):
                src_all[r, pl.ds(k * 16, 16)] = (
                    src_all[r, pl.ds(k * 16, 16)] * 2 + c)
            return carry
        lax.fori_loop(0, NCH, off_body, 0)

        plsc.subcore_barrier()

        def gather_desc(i, b):
            return pltpu.make_async_copy(
                xs_hbm.at[src_all.at[i]], rows[b], gsem)

        def scatter_desc(i, b):
            return pltpu.make_async_copy(
                rows[b], agg_sh.at[dst_all.at[i]], ssem)

        for b in range(GAHEAD):
            gather_desc(b, b).start()

        def round_body(j, carry):
            for b in range(NBUF):
                i = j * NBUF + b
                gather_desc(i, b).wait()
                pltpu.async_copy(rows[b], agg_sh.at[dst_all.at[i]], ssem,
                                 add=True)
                if with_cnt:
                    @pl.when((i % 2) == c)
                    def _():
                        pltpu.async_copy(ebuf, cnt_sh.at[dst_all.at[i]],
                                         csem, add=True)

                        @pl.when(i >= 2)
                        def _():
                            pltpu.make_async_copy(
                                ebuf, cnt_sh.at[dst_all.at[i]], csem).wait()

                @pl.when(i >= SLAG)
                def _():
                    scatter_desc(i, b).wait()

                @pl.when(i + GAHEAD < NCH)
                def _():
                    gather_desc(i + GAHEAD, (b + GAHEAD) % NBUF).start()
            return carry
        lax.fori_loop(0, NCH // NBUF, round_body, 0)

        for _ in range(SLAG):
            scatter_desc(0, 0).wait()
        if with_cnt:
            pltpu.make_async_copy(ebuf, cnt_sh.at[dst_all.at[0]], csem).wait()

        plsc.subcore_barrier()

        obase = c * NP + rbase
        pltpu.sync_copy(agg_sh.at[pl.ds(rbase, RPT)],
                        agg_out.at[pl.ds(obase, RPT), pl.ds(0, DH)])
        if with_cnt:
            pltpu.sync_copy(cnt_sh.at[pl.ds(rbase, RPT)],
                            cnt_out.at[pl.ds(obase, RPT), pl.ds(0, CNTW)])
    return _agg_body


def _build_sc_agg(with_cnt):
    out_type = [jax.ShapeDtypeStruct((NC * NP, D), _f32)]
    if with_cnt:
        out_type.append(jax.ShapeDtypeStruct((NC * NP, D), _f32))
    scratch = [
        pltpu.VMEM((NCH, CH), jnp.int32),
        pltpu.VMEM((NCH, CH), jnp.int32),
    ]
    scratch += [pltpu.VMEM((CH, DH), _f32) for _ in range(NBUF)]
    scratch += [pltpu.VMEM((ZR, DH), _f32)]
    if with_cnt:
        scratch += [pltpu.VMEM((ZCR, CNTW), _f32),
                    pltpu.VMEM((CH, CNTW), _f32)]
    scratch += [pltpu.SemaphoreType.DMA,
                pltpu.SemaphoreType.DMA]
    if with_cnt:
        scratch += [pltpu.SemaphoreType.DMA]
    scratch += [pltpu.SemaphoreType.DMA,
                pltpu.VMEM_SHARED((NP, DH), _f32)]
    if with_cnt:
        scratch += [pltpu.VMEM_SHARED((NP, CNTW), _f32)]
    return pl.kernel(
        _make_agg_body(with_cnt),
        out_type=tuple(out_type) if with_cnt else out_type[0],
        mesh=plsc.VectorSubcoreMesh(core_axis_name="c", subcore_axis_name="s"),
        compiler_params=pltpu.CompilerParams(use_tc_tiling_on_sc=False),
        scratch_types=scratch,
    )


_sc_agg_cnt = _build_sc_agg(True)
_sc_agg = _build_sc_agg(False)


def _mean_from_parts(agg_ref, cnt_ref):
    cnt = jnp.sum(cnt_ref[0, :, :CNTW] + cnt_ref[1, :, :CNTW],
                  axis=1, keepdims=True)
    inv = 1.0 / jnp.maximum(cnt, 1.0)
    return agg_ref[0, :, :DH] * inv, agg_ref[1, :, :DH] * inv


def _sage_out(m0, m1, xd, wl_ref, bl_ref, wr_ref):
    out = lax.dot_general(m0, wl_ref[:, :DH], (((1,), (1,)), ((), ())),
                          preferred_element_type=_f32)
    out = out + lax.dot_general(m1, wl_ref[:, DH:], (((1,), (1,)), ((), ())),
                                preferred_element_type=_f32)
    out = out + lax.dot_general(xd, wr_ref[...], (((1,), (1,)), ((), ())),
                                preferred_element_type=_f32)
    out = out + bl_ref[...]
    return jnp.maximum(out, 0.0)


def _tc_body(agg_ref, cnt_ref, x_ref, wl_ref, bl_ref, wr_ref, o_ref):
    m0, m1 = _mean_from_parts(agg_ref, cnt_ref)
    o_ref[...] = _sage_out(m0, m1, x_ref[...], wl_ref, bl_ref, wr_ref)


R = 2000


def _tc_layer(agg, cnt, x, Wl, bl, Wr):
    return pl.pallas_call(
        _tc_body,
        grid=(N // R,),
        in_specs=[
            pl.BlockSpec((NC, R, D), lambda i: (0, i, 0)),
            pl.BlockSpec((NC, R, D), lambda i: (0, i, 0)),
            pl.BlockSpec((R, D), lambda i: (i, 0)),
            pl.BlockSpec((D, D), lambda i: (0, 0)),
            pl.BlockSpec((1, D), lambda i: (0, 0)),
            pl.BlockSpec((D, D), lambda i: (0, 0)),
        ],
        out_specs=pl.BlockSpec((R, D), lambda i: (i, 0)),
        out_shape=jax.ShapeDtypeStruct((N, D), _f32),
    )(agg, cnt, x, Wl, bl, Wr)


def kernel(x, edge_index, batch, Wl1, bl1, Wr1, Wl2, bl2, Wr2):
    src = edge_index[0].reshape(E // CH, CH)
    dst = edge_index[1].reshape(E // CH, CH)
    agg1, cnt1 = _sc_agg_cnt(x.reshape(NC * N, DH), src, dst)
    agg1 = agg1.reshape(NC, NP, D)
    cnt1 = cnt1.reshape(NC, NP, D)
    h = _tc_layer(agg1, cnt1, x, Wl1, bl1.reshape(1, D), Wr1)
    agg2 = _sc_agg(h.reshape(NC * N, DH), src, dst).reshape(NC, NP, D)
    out = _tc_layer(agg2, cnt1, h, Wl2, bl2.reshape(1, D), Wr2)
    return out
